# Initial kernel scaffold; baseline (speedup 1.0000x reference)
import jax
import jax.numpy as jnp
from jax.experimental import pallas as pl


def kernel(edge_list, node_embeddings, W1, b1, W2, b2):
    raise NotImplementedError("write your pallas kernel here")


# speedup vs baseline: 2.7251x; 2.7251x over previous
import functools

import jax
import jax.numpy as jnp
from jax import lax
from jax.experimental import pallas as pl
from jax.experimental.pallas import tpu as pltpu
from jax.experimental.pallas import tpu_sc as plsc

_D = 128
_LANES = 16
_CH = 128


def _tables_body(emb_ref, w1a_ref, w1b_ref, b1_ref, ta_ref, tb_ref):
    emb = emb_ref[...]
    ta_ref[...] = (
        jnp.dot(emb, w1a_ref[...], preferred_element_type=jnp.float32)
        + b1_ref[...]
    )
    tb_ref[...] = jnp.dot(emb, w1b_ref[...], preferred_element_type=jnp.float32)


def _node_tables(emb, w1a_t, w1b_t, b1_row):
    n = emb.shape[0]
    return pl.pallas_call(
        _tables_body,
        out_shape=(
            jax.ShapeDtypeStruct((n, _D), jnp.float32),
            jax.ShapeDtypeStruct((n, _D), jnp.float32),
        ),
    )(emb, w1a_t, w1b_t, b1_row)


@functools.cache
def _make_sc_scores(e_pad: int):
    info = plsc.get_sparse_core_info()
    nc, ns = info.num_cores, info.num_subcores
    nw = nc * ns
    per_w = e_pad // nw
    n_chunks = per_w // _CH
    mesh = plsc.VectorSubcoreMesh(core_axis_name="c", subcore_axis_name="s")

    @functools.partial(
        pl.kernel,
        mesh=mesh,
        compiler_params=pltpu.CompilerParams(needs_layout_passes=False),
        out_type=jax.ShapeDtypeStruct((e_pad,), jnp.float32),
        scratch_types=[
            pltpu.VMEM((_CH,), jnp.int32),
            pltpu.VMEM((_CH,), jnp.int32),
            pltpu.VMEM((_CH, _D), jnp.float32),
            pltpu.VMEM((_CH, _D), jnp.float32),
            pltpu.VMEM((_CH,), jnp.float32),
            pltpu.VMEM((_D,), jnp.float32),
            pltpu.VMEM((_LANES, _LANES), jnp.float32),
            pltpu.SemaphoreType.DMA,
            pltpu.SemaphoreType.DMA,
        ],
    )
    def sc_scores(ta_hbm, tb_hbm, src_hbm, dst_hbm, w2_hbm, out_hbm,
                  src_v, dst_v, a_v, b_v, s_v, w2_v, m_v, sem_a, sem_b):
        wid = lax.axis_index("s") * nc + lax.axis_index("c")
        base = wid * per_w
        pltpu.sync_copy(w2_hbm, w2_v)
        w2regs = [w2_v[pl.ds(_LANES * c, _LANES)] for c in range(_D // _LANES)]
        lanes = lax.iota(jnp.int32, 16)

        def chunk_body(j, carry):
            off = pl.multiple_of(base + j * _CH, 128)
            pltpu.sync_copy(src_hbm.at[pl.ds(off, _CH)], src_v)
            pltpu.sync_copy(dst_hbm.at[pl.ds(off, _CH)], dst_v)
            cp_a = pltpu.async_copy(ta_hbm.at[src_v], a_v, sem_a)
            cp_b = pltpu.async_copy(tb_hbm.at[dst_v], b_v, sem_b)
            cp_a.wait()
            cp_b.wait()

            def grp_body(g, carry2):
                for e in range(_LANES):
                    i = g * _LANES + e
                    acc = jnp.zeros((_LANES,), jnp.float32)
                    for c in range(_D // _LANES):
                        a = a_v[i, pl.ds(_LANES * c, _LANES)]
                        b = b_v[i, pl.ds(_LANES * c, _LANES)]
                        acc = acc + jnp.maximum(a + b, 0.0) * w2regs[c]
                    m_v[e, :] = acc
                tot = jnp.zeros((_LANES,), jnp.float32)
                for c in range(_LANES):
                    col = jnp.full((_LANES,), c, jnp.int32)
                    tot = tot + plsc.load_gather(m_v, [lanes, col])
                s_v[pl.ds(g * _LANES, _LANES)] = tot
                return carry2

            lax.fori_loop(0, _CH // _LANES, grp_body, 0)
            pltpu.sync_copy(s_v, out_hbm.at[pl.ds(off, _CH)])
            return carry

        lax.fori_loop(0, n_chunks, chunk_body, 0)

    return sc_scores


def _softmax_body(x_ref, o_ref):
    x = x_ref[...]
    m = jnp.max(x)
    e = jnp.exp(x - m)
    o_ref[...] = e / jnp.sum(e)


def _softmax(x2d):
    return pl.pallas_call(
        _softmax_body,
        out_shape=jax.ShapeDtypeStruct(x2d.shape, jnp.float32),
    )(x2d)


def kernel(edge_list, node_embeddings, W1, b1, W2, b2):
    n_edges = edge_list.shape[0]
    d = node_embeddings.shape[1]
    src = edge_list[:, 0].astype(jnp.int32)
    dst = edge_list[:, 1].astype(jnp.int32)

    stride = 32 * _CH
    e_pad = ((n_edges + stride - 1) // stride) * stride
    pad = e_pad - n_edges
    if pad:
        src = jnp.concatenate([src, jnp.zeros((pad,), jnp.int32)])
        dst = jnp.concatenate([dst, jnp.zeros((pad,), jnp.int32)])

    w1a_t = W1[:, :d].T
    w1b_t = W1[:, d:].T
    ta, tb = _node_tables(node_embeddings, w1a_t, w1b_t, b1.reshape(1, d))

    scores = _make_sc_scores(e_pad)(ta, tb, src, dst, W2.reshape(d))
    scores = scores[:n_edges]
    probs = _softmax(scores.reshape(n_edges // _D, _D)).reshape(n_edges)
    return probs


# speedup vs baseline: 3.0438x; 1.1170x over previous
import functools

import jax
import jax.numpy as jnp
from jax import lax
from jax.experimental import pallas as pl
from jax.experimental.pallas import tpu as pltpu
from jax.experimental.pallas import tpu_sc as plsc

_D = 128
_LANES = 16
_CH = 128


def _tables_body(emb_ref, w1a_ref, w1b_ref, b1_ref, ta_ref, tb_ref):
    emb = emb_ref[...]
    ta_ref[...] = (
        jnp.dot(emb, w1a_ref[...], preferred_element_type=jnp.float32)
        + b1_ref[...]
    )
    tb_ref[...] = jnp.dot(emb, w1b_ref[...], preferred_element_type=jnp.float32)


def _node_tables(emb, w1a_t, w1b_t, b1_row):
    n = emb.shape[0]
    return pl.pallas_call(
        _tables_body,
        out_shape=(
            jax.ShapeDtypeStruct((n, _D), jnp.float32),
            jax.ShapeDtypeStruct((n, _D), jnp.float32),
        ),
    )(emb, w1a_t, w1b_t, b1_row)


@functools.cache
def _make_sc_scores(e_pad: int):
    info = plsc.get_sparse_core_info()
    nc, ns = info.num_cores, info.num_subcores
    nw = nc * ns
    per_w = e_pad // nw
    n_chunks = per_w // _CH
    n_pairs = n_chunks // 2
    mesh = plsc.VectorSubcoreMesh(core_axis_name="c", subcore_axis_name="s")

    @functools.partial(
        pl.kernel,
        mesh=mesh,
        compiler_params=pltpu.CompilerParams(needs_layout_passes=False),
        out_type=jax.ShapeDtypeStruct((e_pad,), jnp.float32),
        scratch_types=[
            pltpu.VMEM((per_w,), jnp.int32),
            pltpu.VMEM((per_w,), jnp.int32),
            pltpu.VMEM((_CH, _D), jnp.float32),
            pltpu.VMEM((_CH, _D), jnp.float32),
            pltpu.VMEM((_CH, _D), jnp.float32),
            pltpu.VMEM((_CH, _D), jnp.float32),
            pltpu.VMEM((per_w,), jnp.float32),
            pltpu.VMEM((_D,), jnp.float32),
            pltpu.VMEM((_LANES, _LANES), jnp.float32),
            pltpu.SemaphoreType.DMA,
            pltpu.SemaphoreType.DMA,
            pltpu.SemaphoreType.DMA,
            pltpu.SemaphoreType.DMA,
        ],
    )
    def sc_scores(ta_hbm, tb_hbm, src_hbm, dst_hbm, w2_hbm, out_hbm,
                  srcs, dsts, a0_v, b0_v, a1_v, b1_v, s_all, w2_v, m_v,
                  sa0, sb0, sa1, sb1):
        wid = lax.axis_index("s") * nc + lax.axis_index("c")
        base = pl.multiple_of(wid * per_w, _CH)
        pltpu.sync_copy(w2_hbm, w2_v)
        pltpu.sync_copy(src_hbm.at[pl.ds(base, per_w)], srcs)
        pltpu.sync_copy(dst_hbm.at[pl.ds(base, per_w)], dsts)
        w2regs = [w2_v[pl.ds(_LANES * c, _LANES)] for c in range(_D // _LANES)]
        lanes = lax.iota(jnp.int32, 16)
        slots = ((a0_v, b0_v, sa0, sb0), (a1_v, b1_v, sa1, sb1))

        def issue(j, slot):
            a_v, b_v, sa, sb = slots[slot]
            off = pl.multiple_of(j * _CH, _CH)
            pltpu.make_async_copy(ta_hbm.at[srcs.at[pl.ds(off, _CH)]], a_v, sa).start()
            pltpu.make_async_copy(tb_hbm.at[dsts.at[pl.ds(off, _CH)]], b_v, sb).start()

        def wait(slot):
            a_v, b_v, sa, sb = slots[slot]
            pltpu.make_async_copy(ta_hbm.at[srcs.at[pl.ds(0, _CH)]], a_v, sa).wait()
            pltpu.make_async_copy(tb_hbm.at[dsts.at[pl.ds(0, _CH)]], b_v, sb).wait()

        def compute(j, slot):
            a_v, b_v, _, _ = slots[slot]
            s_off = pl.multiple_of(j * _CH, _CH)

            def grp_body(g, carry2):
                for e in range(_LANES):
                    i = g * _LANES + e
                    acc0 = jnp.maximum(a_v[i, pl.ds(0, _LANES)]
                                       + b_v[i, pl.ds(0, _LANES)], 0.0) * w2regs[0]
                    acc1 = jnp.maximum(a_v[i, pl.ds(_LANES, _LANES)]
                                       + b_v[i, pl.ds(_LANES, _LANES)], 0.0) * w2regs[1]
                    for c in range(2, _D // _LANES, 2):
                        acc0 = acc0 + jnp.maximum(
                            a_v[i, pl.ds(_LANES * c, _LANES)]
                            + b_v[i, pl.ds(_LANES * c, _LANES)], 0.0) * w2regs[c]
                        acc1 = acc1 + jnp.maximum(
                            a_v[i, pl.ds(_LANES * (c + 1), _LANES)]
                            + b_v[i, pl.ds(_LANES * (c + 1), _LANES)], 0.0) * w2regs[c + 1]
                    m_v[e, :] = acc0 + acc1
                tot = jnp.zeros((_LANES,), jnp.float32)
                for c in range(_LANES):
                    col = jnp.full((_LANES,), c, jnp.int32)
                    tot = tot + plsc.load_gather(m_v, [lanes, col])
                s_all[pl.ds(s_off + g * _LANES, _LANES)] = tot
                return carry2

            lax.fori_loop(0, _CH // _LANES, grp_body, 0)

        issue(0, 0)

        def pair_body(jj, carry):
            j0 = jj * 2
            issue(j0 + 1, 1)
            wait(0)
            compute(j0, 0)
            pl.when(jj + 1 < n_pairs)(lambda: issue(j0 + 2, 0))
            wait(1)
            compute(j0 + 1, 1)
            return carry

        lax.fori_loop(0, n_pairs, pair_body, 0)
        pltpu.sync_copy(s_all, out_hbm.at[pl.ds(base, per_w)])

    return sc_scores


def _softmax_body(x_ref, o_ref):
    x = x_ref[...]
    m = jnp.max(x)
    e = jnp.exp(x - m)
    o_ref[...] = e / jnp.sum(e)


def _softmax(x2d):
    return pl.pallas_call(
        _softmax_body,
        out_shape=jax.ShapeDtypeStruct(x2d.shape, jnp.float32),
    )(x2d)


def kernel(edge_list, node_embeddings, W1, b1, W2, b2):
    n_edges = edge_list.shape[0]
    d = node_embeddings.shape[1]
    src = edge_list[:, 0].astype(jnp.int32)
    dst = edge_list[:, 1].astype(jnp.int32)

    stride = 32 * 2 * _CH
    e_pad = ((n_edges + stride - 1) // stride) * stride
    pad = e_pad - n_edges
    if pad:
        src = jnp.concatenate([src, jnp.zeros((pad,), jnp.int32)])
        dst = jnp.concatenate([dst, jnp.zeros((pad,), jnp.int32)])

    w1a_t = W1[:, :d].T
    w1b_t = W1[:, d:].T
    ta, tb = _node_tables(node_embeddings, w1a_t, w1b_t, b1.reshape(1, d))

    scores = _make_sc_scores(e_pad)(ta, tb, src, dst, W2.reshape(d))
    scores = scores[:n_edges]
    probs = _softmax(scores.reshape(n_edges // _D, _D)).reshape(n_edges)
    return probs


# speedup vs baseline: 4.6573x; 1.5301x over previous
import functools

import jax
import jax.numpy as jnp
from jax import lax
from jax.experimental import pallas as pl
from jax.experimental.pallas import tpu as pltpu
from jax.experimental.pallas import tpu_sc as plsc

_D = 128
_LANES = 16
_CH = 128


def _tables_body(emb_ref, w1a_ref, w1b_ref, b1_ref, ta_ref, tb_ref):
    emb = emb_ref[...]
    ta_ref[...] = (
        jnp.dot(emb, w1a_ref[...], preferred_element_type=jnp.float32)
        + b1_ref[...]
    ).astype(jnp.bfloat16)
    tb_ref[...] = jnp.dot(
        emb, w1b_ref[...], preferred_element_type=jnp.float32
    ).astype(jnp.bfloat16)


def _node_tables(emb, w1a_t, w1b_t, b1_row):
    n = emb.shape[0]
    return pl.pallas_call(
        _tables_body,
        out_shape=(
            jax.ShapeDtypeStruct((n, _D), jnp.bfloat16),
            jax.ShapeDtypeStruct((n, _D), jnp.bfloat16),
        ),
    )(emb, w1a_t, w1b_t, b1_row)


@functools.cache
def _make_sc_scores(e_pad: int):
    info = plsc.get_sparse_core_info()
    nc, ns = info.num_cores, info.num_subcores
    nw = nc * ns
    per_w = e_pad // nw
    n_chunks = per_w // _CH
    n_pairs = n_chunks // 2
    mesh = plsc.VectorSubcoreMesh(core_axis_name="c", subcore_axis_name="s")

    @functools.partial(
        pl.kernel,
        mesh=mesh,
        compiler_params=pltpu.CompilerParams(
            needs_layout_passes=False, use_tc_tiling_on_sc=False),
        out_type=jax.ShapeDtypeStruct((e_pad,), jnp.float32),
        scratch_types=[
            pltpu.VMEM((per_w,), jnp.int32),
            pltpu.VMEM((per_w,), jnp.int32),
            pltpu.VMEM((_CH, _D // 2), jnp.int32),
            pltpu.VMEM((_CH, _D // 2), jnp.int32),
            pltpu.VMEM((_CH, _D // 2), jnp.int32),
            pltpu.VMEM((_CH, _D // 2), jnp.int32),
            pltpu.VMEM((per_w,), jnp.float32),
            pltpu.VMEM((_D,), jnp.float32),
            pltpu.VMEM((_LANES, _LANES), jnp.float32),
            pltpu.SemaphoreType.DMA,
            pltpu.SemaphoreType.DMA,
            pltpu.SemaphoreType.DMA,
            pltpu.SemaphoreType.DMA,
        ],
    )
    def sc_scores(ta_hbm, tb_hbm, src_hbm, dst_hbm, w2_hbm, out_hbm,
                  srcs, dsts, a0_v, b0_v, a1_v, b1_v, s_all, w2_v, m_v,
                  sa0, sb0, sa1, sb1):
        wid = lax.axis_index("s") * nc + lax.axis_index("c")
        base = pl.multiple_of(wid * per_w, _CH)
        pltpu.sync_copy(w2_hbm, w2_v)
        pltpu.sync_copy(src_hbm.at[pl.ds(base, per_w)], srcs)
        pltpu.sync_copy(dst_hbm.at[pl.ds(base, per_w)], dsts)
        w2regs = [w2_v[pl.ds(_LANES * c, _LANES)] for c in range(_D // _LANES)]
        lanes = lax.iota(jnp.int32, 16)
        slots = ((a0_v, b0_v, sa0, sb0), (a1_v, b1_v, sa1, sb1))

        def issue(j, slot):
            a_v, b_v, sa, sb = slots[slot]
            off = pl.multiple_of(j * _CH, _CH)
            pltpu.make_async_copy(ta_hbm.at[srcs.at[pl.ds(off, _CH)]], a_v, sa).start()
            pltpu.make_async_copy(tb_hbm.at[dsts.at[pl.ds(off, _CH)]], b_v, sb).start()

        def wait(slot):
            a_v, b_v, sa, sb = slots[slot]
            pltpu.make_async_copy(ta_hbm.at[srcs.at[pl.ds(0, _CH)]], a_v, sa).wait()
            pltpu.make_async_copy(tb_hbm.at[dsts.at[pl.ds(0, _CH)]], b_v, sb).wait()

        def compute(j, slot):
            a_v, b_v, _, _ = slots[slot]
            s_off = pl.multiple_of(j * _CH, _CH)

            def grp_body(g, carry2):
                zero = jnp.zeros((2 * _LANES,), jnp.bfloat16)
                for e in range(_LANES):
                    i = g * _LANES + e
                    acc0 = jnp.zeros((_LANES,), jnp.float32)
                    acc1 = jnp.zeros((_LANES,), jnp.float32)
                    for p in range(_D // (2 * _LANES)):
                        a = plsc.bitcast(
                            a_v[i, pl.ds(_LANES * p, _LANES)], jnp.bfloat16)
                        b = plsc.bitcast(
                            b_v[i, pl.ds(_LANES * p, _LANES)], jnp.bfloat16)
                        r = jnp.maximum(a + b, zero)
                        lo, hi = plsc.unpack(r, format=plsc.PackFormat.INTERLEAVED)
                        acc0 = acc0 + lo * w2regs[2 * p]
                        acc1 = acc1 + hi * w2regs[2 * p + 1]
                    m_v[e, :] = acc0 + acc1
                tot = jnp.zeros((_LANES,), jnp.float32)
                for c in range(_LANES):
                    col = jnp.full((_LANES,), c, jnp.int32)
                    tot = tot + plsc.load_gather(m_v, [lanes, col])
                s_all[pl.ds(s_off + g * _LANES, _LANES)] = tot
                return carry2

            lax.fori_loop(0, _CH // _LANES, grp_body, 0)

        issue(0, 0)

        def pair_body(jj, carry):
            j0 = jj * 2
            issue(j0 + 1, 1)
            wait(0)
            compute(j0, 0)
            pl.when(jj + 1 < n_pairs)(lambda: issue(j0 + 2, 0))
            wait(1)
            compute(j0 + 1, 1)
            return carry

        lax.fori_loop(0, n_pairs, pair_body, 0)
        pltpu.sync_copy(s_all, out_hbm.at[pl.ds(base, per_w)])

    return sc_scores


def _softmax_body(x_ref, o_ref):
    x = x_ref[...]
    m = jnp.max(x)
    e = jnp.exp(x - m)
    o_ref[...] = e / jnp.sum(e)


def _softmax(x2d):
    return pl.pallas_call(
        _softmax_body,
        out_shape=jax.ShapeDtypeStruct(x2d.shape, jnp.float32),
    )(x2d)


def kernel(edge_list, node_embeddings, W1, b1, W2, b2):
    n_edges = edge_list.shape[0]
    d = node_embeddings.shape[1]
    src = edge_list[:, 0].astype(jnp.int32)
    dst = edge_list[:, 1].astype(jnp.int32)

    stride = 32 * 2 * _CH
    e_pad = ((n_edges + stride - 1) // stride) * stride
    pad = e_pad - n_edges
    if pad:
        src = jnp.concatenate([src, jnp.zeros((pad,), jnp.int32)])
        dst = jnp.concatenate([dst, jnp.zeros((pad,), jnp.int32)])

    w1a_t = W1[:, :d].T
    w1b_t = W1[:, d:].T
    ta, tb = _node_tables(node_embeddings, w1a_t, w1b_t, b1.reshape(1, d))
    n = ta.shape[0]
    ta_w = jax.lax.bitcast_convert_type(ta.reshape(n, d // 2, 2), jnp.int32)
    tb_w = jax.lax.bitcast_convert_type(tb.reshape(n, d // 2, 2), jnp.int32)

    w2_arr = W2.reshape(d // 32, 16, 2).transpose(0, 2, 1).reshape(d)
    scores = _make_sc_scores(e_pad)(ta_w, tb_w, src, dst, w2_arr)
    scores = scores[:n_edges]
    probs = _softmax(scores.reshape(n_edges // _D, _D)).reshape(n_edges)
    return probs
